# baseline (speedup 1.0000x reference)
import functools

import jax
import jax.numpy as jnp
from jax import lax
from jax.experimental import pallas as pl
from jax.experimental.pallas import tpu as pltpu
from jax.experimental.pallas import tpu_sc as plsc

N_NODES = 10000
N_EDGES = 320000
HID = 128
EPS = 1e-7
DEN_EPS = 1e-16
LN_EPS = 1e-5

ROWS = 1000
GRID = N_NODES // ROWS


def _ln(h, g, b):
    mu = jnp.mean(h, axis=-1, keepdims=True)
    var = jnp.mean((h - mu) ** 2, axis=-1, keepdims=True)
    return (h - mu) / jnp.sqrt(var + LN_EPS) * g + b


def _tables(cur, t):
    y = jnp.maximum(cur, 0.0) + EPS
    w = jnp.exp(y * t)
    return y * w, w


def _fc_body(x_ref, w_ref, b_ref, t_ref, h_ref, qw_ref):
    h = jnp.dot(x_ref[...], w_ref[...], preferred_element_type=jnp.float32)
    h = jnp.maximum(h + b_ref[...], 0.0)
    h_ref[...] = h
    q, w = _tables(h, t_ref[...])
    qw_ref[0] = q
    qw_ref[1] = w


def _fc_tables(x, wfc, bfc, t):
    return pl.pallas_call(
        _fc_body,
        grid=(GRID,),
        in_specs=[
            pl.BlockSpec((ROWS, 1024), lambda i: (i, 0)),
            pl.BlockSpec((1024, HID), lambda i: (0, 0)),
            pl.BlockSpec((1, HID), lambda i: (0, 0)),
            pl.BlockSpec((1, 1), lambda i: (0, 0)),
        ],
        out_specs=[
            pl.BlockSpec((ROWS, HID), lambda i: (i, 0)),
            pl.BlockSpec((2, ROWS, HID), lambda i: (0, i, 0)),
        ],
        out_shape=[
            jax.ShapeDtypeStruct((N_NODES, HID), jnp.float32),
            jax.ShapeDtypeStruct((2, N_NODES, HID), jnp.float32),
        ],
    )(x, wfc, bfc.reshape(1, HID), t.reshape(1, 1))


NPAD = 10240


CH = 128
KB = 1
NS = 16
EPAD = 327680
DUMP = 10200


def _build_seg():
    mesh = plsc.VectorSubcoreMesh(core_axis_name="c", subcore_axis_name="s")
    EPW = EPAD // NS
    CPT = EPW // CH
    NSUPER = CPT // KB
    RPS = NPAD // NS
    ZR = 32
    IBYTES = KB * CH * 4
    RBYTES = CH * HID * 4

    @functools.partial(
        pl.kernel,
        mesh=mesh,
        out_type=jax.ShapeDtypeStruct((2 * NPAD, HID), jnp.float32),
        scratch_types=[
            pltpu.VMEM((KB * CH,), jnp.int32),
            pltpu.VMEM((KB * CH,), jnp.int32),
            pltpu.VMEM((KB * CH,), jnp.int32),
            pltpu.VMEM((KB * CH,), jnp.int32),
            pltpu.VMEM((2 * KB, CH), jnp.int32),
            pltpu.VMEM((2 * KB, CH), jnp.int32),
            pltpu.VMEM((2, KB, CH, HID), jnp.float32),
            pltpu.VMEM((ZR, HID), jnp.float32),
            pltpu.VMEM_SHARED((NPAD, HID), jnp.float32),
            pltpu.SemaphoreType.DMA,
            pltpu.SemaphoreType.DMA,
            pltpu.SemaphoreType.DMA,
        ],
    )
    def seg(qw_hbm, src_hbm, dst_hbm, out_hbm, slin0, slin1, dlin0, dlin1,
            sblk, dblk, rows, buf, acc, isem, gsem, ssem):
        slin = (slin0, slin1)
        dlin = (dlin0, dlin1)
        c = lax.axis_index("c")
        s = lax.axis_index("s")
        coff = c * N_NODES
        ooff = c * NPAD

        def zrow(r, _):
            for j in range(HID // 16):
                buf[r, pl.ds(j * 16, 16)] = jnp.zeros((16,), jnp.float32)
            return _
        lax.fori_loop(0, ZR, zrow, None)

        def zacc(k, _):
            pltpu.sync_copy(buf, acc.at[pl.ds(s * RPS + k * ZR, ZR)])
            return _
        lax.fori_loop(0, RPS // ZR, zacc, None)
        plsc.subcore_barrier()

        e0 = s * EPW
        EB = KB * CH

        def load_idx(g, p):
            sl = pl.ds(e0 + g * EB, EB)
            pltpu.async_copy(src_hbm.at[sl], slin[p], isem)
            pltpu.async_copy(dst_hbm.at[sl], dlin[p], isem)

        def wait_idx(g, p):
            sl = pl.ds(e0 + g * EB, EB)
            pltpu.make_async_copy(src_hbm.at[sl], slin[p], isem).wait()
            pltpu.make_async_copy(dst_hbm.at[sl], dlin[p], isem).wait()

        def wait_scatters(p):
            for b in range(KB):
                r = p * KB + b
                pltpu.make_async_copy(rows.at[p, b], acc.at[dblk.at[r]],
                                      ssem).wait()

        load_idx(0, 0)

        def block(i, g, p):
            @pl.when(g >= 2)
            def _():
                wait_scatters(p)
            wait_idx(g, p)
            for b in range(KB):
                r = p * KB + b
                for k in range(CH // 16):
                    f = pl.ds(b * CH + k * 16, 16)
                    sl = pl.ds(k * 16, 16)
                    sblk[r, sl] = slin[p][f] + coff
                    dblk[r, sl] = dlin[p][f]
            @pl.when(g + 1 < NSUPER)
            def _():
                load_idx(g + 1, 1 - p)
            handles = [pltpu.async_copy(qw_hbm.at[sblk.at[p * KB + b]],
                                        rows.at[p, b], gsem)
                       for b in range(KB)]
            for h in handles:
                h.wait()
            for b in range(KB):
                pltpu.async_copy(rows.at[p, b], acc.at[dblk.at[p * KB + b]],
                                 ssem, add=True)

        def two_blocks(i, _):
            block(i, 2 * i, 0)
            block(i, 2 * i + 1, 1)
            return _
        lax.fori_loop(0, NSUPER // 2, two_blocks, None)
        if NSUPER % 2:
            block(0, jnp.int32(NSUPER - 1), 0)
        wait_scatters(0)
        wait_scatters(1)

        plsc.subcore_barrier()

        r0 = s * RPS
        pltpu.sync_copy(acc.at[pl.ds(r0, RPS)], out_hbm.at[pl.ds(ooff + r0, RPS)])

    return seg


_seg_cache = []


def _seg_call(qw2, src, dst):
    if not _seg_cache:
        _seg_cache.append(_build_seg())
    return _seg_cache[0](qw2, src, dst)


def _mlp_body(has_norm, make_tables,
              num_ref, den_ref, cur_ref, w1_ref, b1_ref, g1_ref, be1_ref,
              w2_ref, b2_ref, *rest):
    if has_norm:
        lng_ref, lnb_ref = rest[0], rest[1]
        rest = rest[2:]
    if make_tables:
        t_ref = rest[0]
        rest = rest[1:]
    cur_out_ref = rest[0]

    cur = cur_ref[...]
    agg = num_ref[0] / (den_ref[0] + DEN_EPS) + cur
    h1 = jnp.dot(agg, w1_ref[...], preferred_element_type=jnp.float32) + b1_ref[...]
    h1 = jnp.maximum(_ln(h1, g1_ref[...], be1_ref[...]), 0.0)
    co = jnp.dot(h1, w2_ref[...], preferred_element_type=jnp.float32) + b2_ref[...]
    if has_norm:
        co = cur + jnp.maximum(_ln(co, lng_ref[...], lnb_ref[...]), 0.0)
    cur_out_ref[...] = co
    if make_tables:
        qw_ref = rest[1]
        q, w = _tables(co, t_ref[...])
        qw_ref[0] = q
        qw_ref[1] = w


def _mlp_layer(seg3, cur, c, has_norm, t_next):
    make_tables = t_next is not None
    ins = [seg3, seg3, cur, c['W1'], c['b1'].reshape(1, -1),
           c['g1'].reshape(1, -1), c['be1'].reshape(1, -1),
           c['W2'], c['b2'].reshape(1, -1)]
    in_specs = [
        pl.BlockSpec((1, ROWS, HID), lambda i: (0, i, 0)),
        pl.BlockSpec((1, ROWS, HID), lambda i: (1, i, 0)),
        pl.BlockSpec((ROWS, HID), lambda i: (i, 0)),
        pl.BlockSpec((HID, 2 * HID), lambda i: (0, 0)),
        pl.BlockSpec((1, 2 * HID), lambda i: (0, 0)),
        pl.BlockSpec((1, 2 * HID), lambda i: (0, 0)),
        pl.BlockSpec((1, 2 * HID), lambda i: (0, 0)),
        pl.BlockSpec((2 * HID, HID), lambda i: (0, 0)),
        pl.BlockSpec((1, HID), lambda i: (0, 0)),
    ]
    if has_norm:
        ins += [c['lng'].reshape(1, -1), c['lnb'].reshape(1, -1)]
        in_specs += [pl.BlockSpec((1, HID), lambda i: (0, 0)),
                     pl.BlockSpec((1, HID), lambda i: (0, 0))]
    if make_tables:
        ins += [t_next.reshape(1, 1)]
        in_specs += [pl.BlockSpec((1, 1), lambda i: (0, 0))]

    out_specs = [pl.BlockSpec((ROWS, HID), lambda i: (i, 0))]
    out_shape = [jax.ShapeDtypeStruct((N_NODES, HID), jnp.float32)]
    if make_tables:
        out_specs.append(pl.BlockSpec((2, ROWS, HID), lambda i: (0, i, 0)))
        out_shape.append(jax.ShapeDtypeStruct((2, N_NODES, HID), jnp.float32))

    res = pl.pallas_call(
        functools.partial(_mlp_body, has_norm, make_tables),
        grid=(GRID,),
        in_specs=in_specs,
        out_specs=out_specs,
        out_shape=out_shape,
    )(*ins)
    return res if make_tables else (res[0], None)


def _head_body(h_ref, c1_ref, c2_ref, c3_ref,
               wphi_ref, bphi_ref, wa_ref, ba_ref, wb_ref, bb_ref,
               wc_ref, bc_ref, wrho_ref, brho_ref, wwsi_ref, bwsi_ref,
               wo1_ref, bo1_ref, wo2_ref, bo2_ref, clin_ref,
               wsi_ref, logits_ref, m_ref, s_ref, acc_ref):
    i = pl.program_id(0)

    @pl.when(i == 0)
    def _init():
        m_ref[0, 0] = -1e30
        s_ref[0, 0] = 0.0
        acc_ref[...] = jnp.zeros_like(acc_ref)

    xb = jnp.concatenate([h_ref[...], c1_ref[...], c2_ref[...], c3_ref[...]], axis=1)
    hp = jnp.dot(xb, wphi_ref[...], preferred_element_type=jnp.float32)
    hp = jnp.maximum(hp + bphi_ref[...], 0.0)
    a = jnp.tanh(jnp.dot(hp, wa_ref[...], preferred_element_type=jnp.float32) + ba_ref[...])
    b = jax.nn.sigmoid(jnp.dot(hp, wb_ref[...], preferred_element_type=jnp.float32) + bb_ref[...])
    att = jnp.sum((a * b) * wc_ref[...], axis=1, keepdims=True) + bc_ref[...]

    m_old = m_ref[0, 0]
    m_new = jnp.maximum(m_old, jnp.max(att))
    corr = jnp.exp(m_old - m_new)
    p = jnp.exp(att - m_new)
    s_new = s_ref[0, 0] * corr + jnp.sum(p)
    acc_new = acc_ref[...] * corr + jnp.sum(p * hp, axis=0, keepdims=True)
    m_ref[0, 0] = m_new
    s_ref[0, 0] = s_new
    acc_ref[...] = acc_new

    @pl.when(i == GRID - 1)
    def _tail():
        hpool = acc_new / s_new
        hr = jnp.maximum(jnp.dot(hpool, wrho_ref[...], preferred_element_type=jnp.float32)
                         + brho_ref[...], 0.0)
        wsi = jnp.maximum(jnp.dot(hr, wwsi_ref[...], preferred_element_type=jnp.float32)
                          + bwsi_ref[...], 0.0)
        wsi_ref[...] = wsi
        full = jnp.concatenate([wsi, clin_ref[...]], axis=1)
        l1 = jnp.maximum(jnp.dot(full, wo1_ref[...], preferred_element_type=jnp.float32)
                         + bo1_ref[...], 0.0)
        logits_ref[...] = jnp.dot(l1, wo2_ref[...], preferred_element_type=jnp.float32) \
            + bo2_ref[...]


def _head(h, c1, c2, c3, p, clin):
    d4 = 4 * HID
    full_spec = lambda shape: pl.BlockSpec(shape, lambda i: tuple(0 for _ in shape))
    ins = [h, c1, c2, c3,
           p['Wphi'], p['bphi'].reshape(1, -1), p['Wa'], p['ba'].reshape(1, -1),
           p['Wb'], p['bb'].reshape(1, -1), p['Wc'].reshape(1, -1), p['bc'].reshape(1, 1),
           p['Wrho'], p['brho'].reshape(1, -1), p['Wwsi'], p['bwsi'].reshape(1, -1),
           p['Wo1'], p['bo1'].reshape(1, -1), p['Wo2'], p['bo2'].reshape(1, -1), clin]
    in_specs = [pl.BlockSpec((ROWS, HID), lambda i: (i, 0))] * 4 + [
        full_spec((d4, d4)), full_spec((1, d4)),
        full_spec((d4, d4)), full_spec((1, d4)),
        full_spec((d4, d4)), full_spec((1, d4)),
        full_spec((1, d4)), full_spec((1, 1)),
        full_spec((d4, d4)), full_spec((1, d4)),
        full_spec((d4, 256)), full_spec((1, 256)),
        full_spec((288, HID)), full_spec((1, HID)),
        full_spec((HID, 4)), full_spec((1, 4)),
        full_spec((1, 32)),
    ]
    return pl.pallas_call(
        _head_body,
        grid=(GRID,),
        in_specs=in_specs,
        out_specs=[full_spec((1, 256)), full_spec((1, 4))],
        out_shape=[jax.ShapeDtypeStruct((1, 256), jnp.float32),
                   jax.ShapeDtypeStruct((1, 4), jnp.float32)],
        scratch_shapes=[pltpu.SMEM((1, 1), jnp.float32),
                        pltpu.SMEM((1, 1), jnp.float32),
                        pltpu.VMEM((1, d4), jnp.float32)],
    )(*ins)


def kernel(x, edge_index, clin, params):
    p = params
    npad = EPAD - N_EDGES
    src = jnp.concatenate([edge_index[0], jnp.zeros((npad,), jnp.int32)])
    dst = jnp.concatenate([edge_index[1], jnp.full((npad,), DUMP, jnp.int32)])

    h, qw = _fc_tables(x, p['Wfc'], p['bfc'], p['conv0']['t'])

    seg0 = _seg_call(qw.reshape(2 * N_NODES, HID), src, dst).reshape(2, NPAD, HID)
    cur1, qw1 = _mlp_layer(seg0, h, p['conv0'], has_norm=False, t_next=p['conv1']['t'])

    seg1 = _seg_call(qw1.reshape(2 * N_NODES, HID), src, dst).reshape(2, NPAD, HID)
    cur2, qw2 = _mlp_layer(seg1, cur1, p['conv1'], has_norm=True, t_next=p['conv2']['t'])

    seg2 = _seg_call(qw2.reshape(2 * N_NODES, HID), src, dst).reshape(2, NPAD, HID)
    cur3, _ = _mlp_layer(seg2, cur2, p['conv2'], has_norm=True, t_next=None)

    wsi, logits = _head(h, cur1, cur2, cur3, p, clin)

    hazards = jax.nn.sigmoid(logits)
    S = jnp.cumprod(1.0 - hazards, axis=1)
    Y_hat = jnp.argmax(logits, axis=1)
    return hazards, S, Y_hat, wsi


# speedup vs baseline: 2.4266x; 2.4266x over previous
import functools

import jax
import jax.numpy as jnp
from jax import lax
from jax.experimental import pallas as pl
from jax.experimental.pallas import tpu as pltpu
from jax.experimental.pallas import tpu_sc as plsc

N_NODES = 10000
N_EDGES = 320000
HID = 128
EPS = 1e-7
DEN_EPS = 1e-16
LN_EPS = 1e-5

ROWS = 1000
GRID = N_NODES // ROWS


def _ln(h, g, b):
    mu = jnp.mean(h, axis=-1, keepdims=True)
    var = jnp.mean((h - mu) ** 2, axis=-1, keepdims=True)
    return (h - mu) / jnp.sqrt(var + LN_EPS) * g + b


def _tables(cur, t):
    y = jnp.maximum(cur, 0.0) + EPS
    w = jnp.exp(y * t)
    return y * w, w


def _fc_body(x_ref, w_ref, b_ref, t_ref, h_ref, qw_ref):
    h = jnp.dot(x_ref[...], w_ref[...], preferred_element_type=jnp.float32)
    h = jnp.maximum(h + b_ref[...], 0.0)
    h_ref[...] = h
    q, w = _tables(h, t_ref[...])
    qw_ref[0] = q
    qw_ref[1] = w


def _fc_tables(x, wfc, bfc, t):
    return pl.pallas_call(
        _fc_body,
        grid=(GRID,),
        in_specs=[
            pl.BlockSpec((ROWS, 1024), lambda i: (i, 0)),
            pl.BlockSpec((1024, HID), lambda i: (0, 0)),
            pl.BlockSpec((1, HID), lambda i: (0, 0)),
            pl.BlockSpec((1, 1), lambda i: (0, 0)),
        ],
        out_specs=[
            pl.BlockSpec((ROWS, HID), lambda i: (i, 0)),
            pl.BlockSpec((2, ROWS, HID), lambda i: (0, i, 0)),
        ],
        out_shape=[
            jax.ShapeDtypeStruct((N_NODES, HID), jnp.float32),
            jax.ShapeDtypeStruct((2, N_NODES, HID), jnp.float32),
        ],
    )(x, wfc, bfc.reshape(1, HID), t.reshape(1, 1))


NPAD = 10240


CH = 80
KB = 2
NS = 16
EPAD = N_EDGES
DUMP = 10200


def _build_seg():
    mesh = plsc.VectorSubcoreMesh(core_axis_name="c", subcore_axis_name="s")
    EPW = EPAD // NS
    CPT = EPW // CH
    NSUPER = CPT // KB
    RPS = NPAD // NS
    ZR = 32

    @functools.partial(
        pl.kernel,
        mesh=mesh,
        out_type=jax.ShapeDtypeStruct((2 * NPAD, HID), jnp.float32),
        scratch_types=[
            pltpu.VMEM((KB * CH,), jnp.int32),
            pltpu.VMEM((KB * CH,), jnp.int32),
            pltpu.VMEM((KB * CH,), jnp.int32),
            pltpu.VMEM((KB * CH,), jnp.int32),
            pltpu.VMEM((2 * KB, CH), jnp.int32),
            pltpu.VMEM((2 * KB, CH), jnp.int32),
            pltpu.VMEM((2, KB, CH, HID), jnp.float32),
            pltpu.VMEM((ZR, HID), jnp.float32),
            pltpu.VMEM_SHARED((NPAD, HID), jnp.float32),
            pltpu.SemaphoreType.DMA,
            pltpu.SemaphoreType.DMA,
            pltpu.SemaphoreType.DMA,
        ],
    )
    def seg(qw_hbm, src_hbm, dst_hbm, out_hbm, slin0, slin1, dlin0, dlin1,
            sblk, dblk, rows, buf, acc, isem, gsem, ssem):
        slin = (slin0, slin1)
        dlin = (dlin0, dlin1)
        c = lax.axis_index("c")
        s = lax.axis_index("s")
        coff = c * N_NODES
        ooff = c * NPAD

        def zrow(r, _):
            for j in range(HID // 16):
                buf[r, pl.ds(j * 16, 16)] = jnp.zeros((16,), jnp.float32)
            return _
        lax.fori_loop(0, ZR, zrow, None)

        def zacc(k, _):
            pltpu.sync_copy(buf, acc.at[pl.ds(s * RPS + k * ZR, ZR)])
            return _
        lax.fori_loop(0, RPS // ZR, zacc, None)
        plsc.subcore_barrier()

        e0 = s * EPW
        EB = KB * CH

        def load_idx(g, p):
            sl = pl.ds(e0 + g * EB, EB)
            pltpu.async_copy(src_hbm.at[sl], slin[p], isem)
            pltpu.async_copy(dst_hbm.at[sl], dlin[p], isem)

        def wait_idx(g, p):
            sl = pl.ds(e0 + g * EB, EB)
            pltpu.make_async_copy(src_hbm.at[sl], slin[p], isem).wait()
            pltpu.make_async_copy(dst_hbm.at[sl], dlin[p], isem).wait()

        def wait_scatters(p):
            for b in range(KB):
                r = p * KB + b
                pltpu.make_async_copy(rows.at[p, b], acc.at[dblk.at[r]],
                                      ssem).wait()

        def wait_gathers(p):
            for b in range(KB):
                pltpu.make_async_copy(qw_hbm.at[sblk.at[p * KB + b]],
                                      rows.at[p, b], gsem).wait()

        def fire_scatters(p):
            for b in range(KB):
                pltpu.async_copy(rows.at[p, b], acc.at[dblk.at[p * KB + b]],
                                 ssem, add=True)

        load_idx(0, 0)

        def block(i, g, p):
            q = 1 - p
            @pl.when(g >= 2)
            def _():
                wait_scatters(p)
            wait_idx(g, p)
            for b in range(KB):
                r = p * KB + b
                for k in range(CH // 16):
                    f = pl.ds(b * CH + k * 16, 16)
                    sl = pl.ds(k * 16, 16)
                    sblk[r, sl] = slin[p][f] + coff
                    dblk[r, sl] = dlin[p][f]
            @pl.when(g + 1 < NSUPER)
            def _():
                load_idx(g + 1, q)
            for b in range(KB):
                pltpu.async_copy(qw_hbm.at[sblk.at[p * KB + b]],
                                 rows.at[p, b], gsem)
            @pl.when(g >= 1)
            def _():
                wait_gathers(q)
                fire_scatters(q)

        def two_blocks(i, _):
            block(i, 2 * i, 0)
            block(i, 2 * i + 1, 1)
            return _
        lax.fori_loop(0, NSUPER // 2, two_blocks, None)
        if NSUPER % 2:
            block(0, jnp.int32(NSUPER - 1), 0)
        last = (NSUPER - 1) % 2
        wait_gathers(last)
        fire_scatters(last)
        wait_scatters(1 - last)
        wait_scatters(last)

        plsc.subcore_barrier()

        r0 = s * RPS
        pltpu.sync_copy(acc.at[pl.ds(r0, RPS)], out_hbm.at[pl.ds(ooff + r0, RPS)])

    return seg


_seg_cache = []


def _seg_call(qw2, src, dst):
    if not _seg_cache:
        _seg_cache.append(_build_seg())
    return _seg_cache[0](qw2, src, dst)


def _mlp_body(has_norm, make_tables,
              num_ref, den_ref, cur_ref, w1_ref, b1_ref, g1_ref, be1_ref,
              w2_ref, b2_ref, *rest):
    if has_norm:
        lng_ref, lnb_ref = rest[0], rest[1]
        rest = rest[2:]
    if make_tables:
        t_ref = rest[0]
        rest = rest[1:]
    cur_out_ref = rest[0]

    cur = cur_ref[...]
    agg = num_ref[0] / (den_ref[0] + DEN_EPS) + cur
    h1 = jnp.dot(agg, w1_ref[...], preferred_element_type=jnp.float32) + b1_ref[...]
    h1 = jnp.maximum(_ln(h1, g1_ref[...], be1_ref[...]), 0.0)
    co = jnp.dot(h1, w2_ref[...], preferred_element_type=jnp.float32) + b2_ref[...]
    if has_norm:
        co = cur + jnp.maximum(_ln(co, lng_ref[...], lnb_ref[...]), 0.0)
    cur_out_ref[...] = co
    if make_tables:
        qw_ref = rest[1]
        q, w = _tables(co, t_ref[...])
        qw_ref[0] = q
        qw_ref[1] = w


def _mlp_layer(seg3, cur, c, has_norm, t_next):
    make_tables = t_next is not None
    ins = [seg3, seg3, cur, c['W1'], c['b1'].reshape(1, -1),
           c['g1'].reshape(1, -1), c['be1'].reshape(1, -1),
           c['W2'], c['b2'].reshape(1, -1)]
    in_specs = [
        pl.BlockSpec((1, ROWS, HID), lambda i: (0, i, 0)),
        pl.BlockSpec((1, ROWS, HID), lambda i: (1, i, 0)),
        pl.BlockSpec((ROWS, HID), lambda i: (i, 0)),
        pl.BlockSpec((HID, 2 * HID), lambda i: (0, 0)),
        pl.BlockSpec((1, 2 * HID), lambda i: (0, 0)),
        pl.BlockSpec((1, 2 * HID), lambda i: (0, 0)),
        pl.BlockSpec((1, 2 * HID), lambda i: (0, 0)),
        pl.BlockSpec((2 * HID, HID), lambda i: (0, 0)),
        pl.BlockSpec((1, HID), lambda i: (0, 0)),
    ]
    if has_norm:
        ins += [c['lng'].reshape(1, -1), c['lnb'].reshape(1, -1)]
        in_specs += [pl.BlockSpec((1, HID), lambda i: (0, 0)),
                     pl.BlockSpec((1, HID), lambda i: (0, 0))]
    if make_tables:
        ins += [t_next.reshape(1, 1)]
        in_specs += [pl.BlockSpec((1, 1), lambda i: (0, 0))]

    out_specs = [pl.BlockSpec((ROWS, HID), lambda i: (i, 0))]
    out_shape = [jax.ShapeDtypeStruct((N_NODES, HID), jnp.float32)]
    if make_tables:
        out_specs.append(pl.BlockSpec((2, ROWS, HID), lambda i: (0, i, 0)))
        out_shape.append(jax.ShapeDtypeStruct((2, N_NODES, HID), jnp.float32))

    res = pl.pallas_call(
        functools.partial(_mlp_body, has_norm, make_tables),
        grid=(GRID,),
        in_specs=in_specs,
        out_specs=out_specs,
        out_shape=out_shape,
    )(*ins)
    return res if make_tables else (res[0], None)


def _head_body(h_ref, c1_ref, c2_ref, c3_ref,
               wphi_ref, bphi_ref, wa_ref, ba_ref, wb_ref, bb_ref,
               wc_ref, bc_ref, wrho_ref, brho_ref, wwsi_ref, bwsi_ref,
               wo1_ref, bo1_ref, wo2_ref, bo2_ref, clin_ref,
               wsi_ref, logits_ref, m_ref, s_ref, acc_ref):
    i = pl.program_id(0)

    @pl.when(i == 0)
    def _init():
        m_ref[0, 0] = -1e30
        s_ref[0, 0] = 0.0
        acc_ref[...] = jnp.zeros_like(acc_ref)

    xb = jnp.concatenate([h_ref[...], c1_ref[...], c2_ref[...], c3_ref[...]], axis=1)
    hp = jnp.dot(xb, wphi_ref[...], preferred_element_type=jnp.float32)
    hp = jnp.maximum(hp + bphi_ref[...], 0.0)
    a = jnp.tanh(jnp.dot(hp, wa_ref[...], preferred_element_type=jnp.float32) + ba_ref[...])
    b = jax.nn.sigmoid(jnp.dot(hp, wb_ref[...], preferred_element_type=jnp.float32) + bb_ref[...])
    att = jnp.sum((a * b) * wc_ref[...], axis=1, keepdims=True) + bc_ref[...]

    m_old = m_ref[0, 0]
    m_new = jnp.maximum(m_old, jnp.max(att))
    corr = jnp.exp(m_old - m_new)
    p = jnp.exp(att - m_new)
    s_new = s_ref[0, 0] * corr + jnp.sum(p)
    acc_new = acc_ref[...] * corr + jnp.sum(p * hp, axis=0, keepdims=True)
    m_ref[0, 0] = m_new
    s_ref[0, 0] = s_new
    acc_ref[...] = acc_new

    @pl.when(i == GRID - 1)
    def _tail():
        hpool = acc_new / s_new
        hr = jnp.maximum(jnp.dot(hpool, wrho_ref[...], preferred_element_type=jnp.float32)
                         + brho_ref[...], 0.0)
        wsi = jnp.maximum(jnp.dot(hr, wwsi_ref[...], preferred_element_type=jnp.float32)
                          + bwsi_ref[...], 0.0)
        wsi_ref[...] = wsi
        full = jnp.concatenate([wsi, clin_ref[...]], axis=1)
        l1 = jnp.maximum(jnp.dot(full, wo1_ref[...], preferred_element_type=jnp.float32)
                         + bo1_ref[...], 0.0)
        logits_ref[...] = jnp.dot(l1, wo2_ref[...], preferred_element_type=jnp.float32) \
            + bo2_ref[...]


def _head(h, c1, c2, c3, p, clin):
    d4 = 4 * HID
    full_spec = lambda shape: pl.BlockSpec(shape, lambda i: tuple(0 for _ in shape))
    ins = [h, c1, c2, c3,
           p['Wphi'], p['bphi'].reshape(1, -1), p['Wa'], p['ba'].reshape(1, -1),
           p['Wb'], p['bb'].reshape(1, -1), p['Wc'].reshape(1, -1), p['bc'].reshape(1, 1),
           p['Wrho'], p['brho'].reshape(1, -1), p['Wwsi'], p['bwsi'].reshape(1, -1),
           p['Wo1'], p['bo1'].reshape(1, -1), p['Wo2'], p['bo2'].reshape(1, -1), clin]
    in_specs = [pl.BlockSpec((ROWS, HID), lambda i: (i, 0))] * 4 + [
        full_spec((d4, d4)), full_spec((1, d4)),
        full_spec((d4, d4)), full_spec((1, d4)),
        full_spec((d4, d4)), full_spec((1, d4)),
        full_spec((1, d4)), full_spec((1, 1)),
        full_spec((d4, d4)), full_spec((1, d4)),
        full_spec((d4, 256)), full_spec((1, 256)),
        full_spec((288, HID)), full_spec((1, HID)),
        full_spec((HID, 4)), full_spec((1, 4)),
        full_spec((1, 32)),
    ]
    return pl.pallas_call(
        _head_body,
        grid=(GRID,),
        in_specs=in_specs,
        out_specs=[full_spec((1, 256)), full_spec((1, 4))],
        out_shape=[jax.ShapeDtypeStruct((1, 256), jnp.float32),
                   jax.ShapeDtypeStruct((1, 4), jnp.float32)],
        scratch_shapes=[pltpu.SMEM((1, 1), jnp.float32),
                        pltpu.SMEM((1, 1), jnp.float32),
                        pltpu.VMEM((1, d4), jnp.float32)],
    )(*ins)


def kernel(x, edge_index, clin, params):
    p = params
    npad = EPAD - N_EDGES
    if npad:
        src = jnp.concatenate([edge_index[0], jnp.zeros((npad,), jnp.int32)])
        dst = jnp.concatenate([edge_index[1], jnp.full((npad,), DUMP, jnp.int32)])
    else:
        src = edge_index[0]
        dst = edge_index[1]

    h, qw = _fc_tables(x, p['Wfc'], p['bfc'], p['conv0']['t'])

    seg0 = _seg_call(qw.reshape(2 * N_NODES, HID), src, dst).reshape(2, NPAD, HID)
    cur1, qw1 = _mlp_layer(seg0, h, p['conv0'], has_norm=False, t_next=p['conv1']['t'])

    seg1 = _seg_call(qw1.reshape(2 * N_NODES, HID), src, dst).reshape(2, NPAD, HID)
    cur2, qw2 = _mlp_layer(seg1, cur1, p['conv1'], has_norm=True, t_next=p['conv2']['t'])

    seg2 = _seg_call(qw2.reshape(2 * N_NODES, HID), src, dst).reshape(2, NPAD, HID)
    cur3, _ = _mlp_layer(seg2, cur2, p['conv2'], has_norm=True, t_next=None)

    wsi, logits = _head(h, cur1, cur2, cur3, p, clin)

    hazards = jax.nn.sigmoid(logits)
    S = jnp.cumprod(1.0 - hazards, axis=1)
    Y_hat = jnp.argmax(logits, axis=1)
    return hazards, S, Y_hat, wsi


# speedup vs baseline: 2.4765x; 1.0206x over previous
import functools

import jax
import jax.numpy as jnp
from jax import lax
from jax.experimental import pallas as pl
from jax.experimental.pallas import tpu as pltpu
from jax.experimental.pallas import tpu_sc as plsc

N_NODES = 10000
N_EDGES = 320000
HID = 128
EPS = 1e-7
DEN_EPS = 1e-16
LN_EPS = 1e-5

ROWS = 1000
GRID = N_NODES // ROWS


def _ln(h, g, b):
    mu = jnp.mean(h, axis=-1, keepdims=True)
    var = jnp.mean((h - mu) ** 2, axis=-1, keepdims=True)
    return (h - mu) / jnp.sqrt(var + LN_EPS) * g + b


def _tables(cur, t):
    y = jnp.maximum(cur, 0.0) + EPS
    w = jnp.exp(y * t)
    return y * w, w


def _fc_body(x_ref, w_ref, b_ref, t_ref, h_ref, qw_ref):
    h = jnp.dot(x_ref[...], w_ref[...], preferred_element_type=jnp.float32)
    h = jnp.maximum(h + b_ref[...], 0.0)
    h_ref[...] = h
    q, w = _tables(h, t_ref[...])
    qw_ref[0] = q
    qw_ref[1] = w


def _fc_tables(x, wfc, bfc, t):
    return pl.pallas_call(
        _fc_body,
        grid=(GRID,),
        in_specs=[
            pl.BlockSpec((ROWS, 1024), lambda i: (i, 0)),
            pl.BlockSpec((1024, HID), lambda i: (0, 0)),
            pl.BlockSpec((1, HID), lambda i: (0, 0)),
            pl.BlockSpec((1, 1), lambda i: (0, 0)),
        ],
        out_specs=[
            pl.BlockSpec((ROWS, HID), lambda i: (i, 0)),
            pl.BlockSpec((2, ROWS, HID), lambda i: (0, i, 0)),
        ],
        out_shape=[
            jax.ShapeDtypeStruct((N_NODES, HID), jnp.float32),
            jax.ShapeDtypeStruct((2, N_NODES, HID), jnp.float32),
        ],
    )(x, wfc, bfc.reshape(1, HID), t.reshape(1, 1))


NPAD = 10240


CH = 80
KB = 2
NS = 16
EPAD = N_EDGES
DUMP = 10200


def _build_seg():
    mesh = plsc.VectorSubcoreMesh(core_axis_name="c", subcore_axis_name="s")
    EPW = EPAD // NS
    CPT = EPW // CH
    NSUPER = CPT // KB
    RPS = NPAD // NS
    ZR = 32

    @functools.partial(
        pl.kernel,
        mesh=mesh,
        out_type=jax.ShapeDtypeStruct((2 * NPAD, HID), jnp.float32),
        scratch_types=[
            pltpu.VMEM((KB * CH,), jnp.int32),
            pltpu.VMEM((KB * CH,), jnp.int32),
            pltpu.VMEM((KB * CH,), jnp.int32),
            pltpu.VMEM((KB * CH,), jnp.int32),
            pltpu.VMEM((2 * KB, CH), jnp.int32),
            pltpu.VMEM((2 * KB, CH), jnp.int32),
            pltpu.VMEM((2, KB, CH, HID), jnp.float32),
            pltpu.VMEM((ZR, HID), jnp.float32),
            pltpu.VMEM_SHARED((NPAD, HID), jnp.float32),
            pltpu.SemaphoreType.DMA,
            pltpu.SemaphoreType.DMA,
            pltpu.SemaphoreType.DMA,
        ],
    )
    def seg(qw_hbm, src_hbm, dst_hbm, out_hbm, slin0, slin1, dlin0, dlin1,
            sblk, dblk, rows, buf, acc, isem, gsem, ssem):
        slin = (slin0, slin1)
        dlin = (dlin0, dlin1)
        c = lax.axis_index("c")
        s = lax.axis_index("s")
        coff = c * N_NODES
        ooff = c * NPAD

        def zrow(r, _):
            for j in range(HID // 16):
                buf[r, pl.ds(j * 16, 16)] = jnp.zeros((16,), jnp.float32)
            return _
        lax.fori_loop(0, ZR, zrow, None)

        def zacc(k, _):
            pltpu.sync_copy(buf, acc.at[pl.ds(s * RPS + k * ZR, ZR)])
            return _
        lax.fori_loop(0, RPS // ZR, zacc, None)
        plsc.subcore_barrier()

        e0 = s * EPW
        EB = KB * CH

        def load_idx(g, p):
            sl = pl.ds(e0 + g * EB, EB)
            pltpu.async_copy(src_hbm.at[sl], slin[p], isem)
            pltpu.async_copy(dst_hbm.at[sl], dlin[p], isem)

        def wait_idx(g, p):
            sl = pl.ds(e0 + g * EB, EB)
            pltpu.make_async_copy(src_hbm.at[sl], slin[p], isem).wait()
            pltpu.make_async_copy(dst_hbm.at[sl], dlin[p], isem).wait()

        def wait_scatters(p):
            for b in range(KB):
                r = p * KB + b
                pltpu.make_async_copy(rows.at[p, b], acc.at[dblk.at[r]],
                                      ssem).wait()

        def wait_gathers(p):
            for b in range(KB):
                pltpu.make_async_copy(qw_hbm.at[sblk.at[p * KB + b]],
                                      rows.at[p, b], gsem).wait()

        def fire_scatters(p):
            for b in range(KB):
                pltpu.async_copy(rows.at[p, b], acc.at[dblk.at[p * KB + b]],
                                 ssem, add=True)

        load_idx(0, 0)

        def block(i, g, p):
            q = 1 - p
            @pl.when(g >= 2)
            def _():
                wait_scatters(p)
            wait_idx(g, p)
            for b in range(KB):
                r = p * KB + b
                for k in range(CH // 16):
                    f = pl.ds(b * CH + k * 16, 16)
                    sl = pl.ds(k * 16, 16)
                    sblk[r, sl] = slin[p][f] + coff
                    dblk[r, sl] = dlin[p][f]
            @pl.when(g + 1 < NSUPER)
            def _():
                load_idx(g + 1, q)
            for b in range(KB):
                pltpu.async_copy(qw_hbm.at[sblk.at[p * KB + b]],
                                 rows.at[p, b], gsem)
            @pl.when(g >= 1)
            def _():
                wait_gathers(q)
                fire_scatters(q)

        def two_blocks(i, _):
            block(i, 2 * i, 0)
            block(i, 2 * i + 1, 1)
            return _
        lax.fori_loop(0, NSUPER // 2, two_blocks, None)
        if NSUPER % 2:
            block(0, jnp.int32(NSUPER - 1), 0)
        last = (NSUPER - 1) % 2
        wait_gathers(last)
        fire_scatters(last)
        wait_scatters(1 - last)
        wait_scatters(last)

        plsc.subcore_barrier()

        r0 = s * RPS
        pltpu.sync_copy(acc.at[pl.ds(r0, RPS)], out_hbm.at[pl.ds(ooff + r0, RPS)])

    return seg


_seg_cache = []


def _seg_call(qw2, src, dst):
    if not _seg_cache:
        _seg_cache.append(_build_seg())
    return _seg_cache[0](qw2, src, dst)


def _mlp_body(has_norm, make_tables,
              num_ref, den_ref, cur_ref, w1_ref, b1_ref, g1_ref, be1_ref,
              w2_ref, b2_ref, *rest):
    if has_norm:
        lng_ref, lnb_ref = rest[0], rest[1]
        rest = rest[2:]
    if make_tables:
        t_ref = rest[0]
        rest = rest[1:]
    cur_out_ref = rest[0]

    cur = cur_ref[...]
    agg = num_ref[0] / (den_ref[0] + DEN_EPS) + cur
    h1 = jnp.dot(agg, w1_ref[...], preferred_element_type=jnp.float32) + b1_ref[...]
    h1 = jnp.maximum(_ln(h1, g1_ref[...], be1_ref[...]), 0.0)
    co = jnp.dot(h1, w2_ref[...], preferred_element_type=jnp.float32) + b2_ref[...]
    if has_norm:
        co = cur + jnp.maximum(_ln(co, lng_ref[...], lnb_ref[...]), 0.0)
    cur_out_ref[...] = co
    if make_tables:
        qw_ref = rest[1]
        q, w = _tables(co, t_ref[...])
        qw_ref[0] = q
        qw_ref[1] = w


def _mlp_layer(seg3, cur, c, has_norm, t_next):
    make_tables = t_next is not None
    ins = [seg3, seg3, cur, c['W1'], c['b1'].reshape(1, -1),
           c['g1'].reshape(1, -1), c['be1'].reshape(1, -1),
           c['W2'], c['b2'].reshape(1, -1)]
    in_specs = [
        pl.BlockSpec((1, ROWS, HID), lambda i: (0, i, 0)),
        pl.BlockSpec((1, ROWS, HID), lambda i: (1, i, 0)),
        pl.BlockSpec((ROWS, HID), lambda i: (i, 0)),
        pl.BlockSpec((HID, 2 * HID), lambda i: (0, 0)),
        pl.BlockSpec((1, 2 * HID), lambda i: (0, 0)),
        pl.BlockSpec((1, 2 * HID), lambda i: (0, 0)),
        pl.BlockSpec((1, 2 * HID), lambda i: (0, 0)),
        pl.BlockSpec((2 * HID, HID), lambda i: (0, 0)),
        pl.BlockSpec((1, HID), lambda i: (0, 0)),
    ]
    if has_norm:
        ins += [c['lng'].reshape(1, -1), c['lnb'].reshape(1, -1)]
        in_specs += [pl.BlockSpec((1, HID), lambda i: (0, 0)),
                     pl.BlockSpec((1, HID), lambda i: (0, 0))]
    if make_tables:
        ins += [t_next.reshape(1, 1)]
        in_specs += [pl.BlockSpec((1, 1), lambda i: (0, 0))]

    out_specs = [pl.BlockSpec((ROWS, HID), lambda i: (i, 0))]
    out_shape = [jax.ShapeDtypeStruct((N_NODES, HID), jnp.float32)]
    if make_tables:
        out_specs.append(pl.BlockSpec((2, ROWS, HID), lambda i: (0, i, 0)))
        out_shape.append(jax.ShapeDtypeStruct((2, N_NODES, HID), jnp.float32))

    res = pl.pallas_call(
        functools.partial(_mlp_body, has_norm, make_tables),
        grid=(GRID,),
        in_specs=in_specs,
        out_specs=out_specs,
        out_shape=out_shape,
    )(*ins)
    return res if make_tables else (res[0], None)


def _head_body(num_ref, den_ref, w1_ref, b1_ref, g1_ref, be1_ref,
               w2_ref, b2_ref, lng_ref, lnb_ref,
               h_ref, c1_ref, c2_ref,
               wphi_ref, bphi_ref, wa_ref, ba_ref, wb_ref, bb_ref,
               wc_ref, bc_ref, wrho_ref, brho_ref, wwsi_ref, bwsi_ref,
               wo1_ref, bo1_ref, wo2_ref, bo2_ref, clin_ref,
               wsi_ref, haz_ref, ss_ref, yh_ref, m_ref, s_ref, acc_ref):
    i = pl.program_id(0)

    @pl.when(i == 0)
    def _init():
        m_ref[0, 0] = -1e30
        s_ref[0, 0] = 0.0
        acc_ref[...] = jnp.zeros_like(acc_ref)

    cur = c2_ref[...]
    agg = num_ref[0] / (den_ref[0] + DEN_EPS) + cur
    h1 = jnp.dot(agg, w1_ref[...], preferred_element_type=jnp.float32) + b1_ref[...]
    h1 = jnp.maximum(_ln(h1, g1_ref[...], be1_ref[...]), 0.0)
    co = jnp.dot(h1, w2_ref[...], preferred_element_type=jnp.float32) + b2_ref[...]
    c3 = cur + jnp.maximum(_ln(co, lng_ref[...], lnb_ref[...]), 0.0)

    xb = jnp.concatenate([h_ref[...], c1_ref[...], cur, c3], axis=1)
    hp = jnp.dot(xb, wphi_ref[...], preferred_element_type=jnp.float32)
    hp = jnp.maximum(hp + bphi_ref[...], 0.0)
    a = jnp.tanh(jnp.dot(hp, wa_ref[...], preferred_element_type=jnp.float32) + ba_ref[...])
    b = jax.nn.sigmoid(jnp.dot(hp, wb_ref[...], preferred_element_type=jnp.float32) + bb_ref[...])
    att = jnp.sum((a * b) * wc_ref[...], axis=1, keepdims=True) + bc_ref[...]

    m_old = m_ref[0, 0]
    m_new = jnp.maximum(m_old, jnp.max(att))
    corr = jnp.exp(m_old - m_new)
    p = jnp.exp(att - m_new)
    s_new = s_ref[0, 0] * corr + jnp.sum(p)
    acc_new = acc_ref[...] * corr + jnp.sum(p * hp, axis=0, keepdims=True)
    m_ref[0, 0] = m_new
    s_ref[0, 0] = s_new
    acc_ref[...] = acc_new

    @pl.when(i == GRID - 1)
    def _tail():
        hpool = acc_new / s_new
        hr = jnp.maximum(jnp.dot(hpool, wrho_ref[...], preferred_element_type=jnp.float32)
                         + brho_ref[...], 0.0)
        wsi = jnp.maximum(jnp.dot(hr, wwsi_ref[...], preferred_element_type=jnp.float32)
                          + bwsi_ref[...], 0.0)
        wsi_ref[...] = wsi
        full = jnp.concatenate([wsi, clin_ref[...]], axis=1)
        l1 = jnp.maximum(jnp.dot(full, wo1_ref[...], preferred_element_type=jnp.float32)
                         + bo1_ref[...], 0.0)
        logits = jnp.dot(l1, wo2_ref[...], preferred_element_type=jnp.float32) \
            + bo2_ref[...]
        hz = jax.nn.sigmoid(logits)
        haz_ref[...] = hz
        q = 1.0 - hz
        s0 = q[:, 0:1]
        s1 = s0 * q[:, 1:2]
        s2 = s1 * q[:, 2:3]
        s3 = s2 * q[:, 3:4]
        ss_ref[...] = jnp.concatenate([s0, s1, s2, s3], axis=1)
        mx = jnp.max(logits)
        ids = lax.broadcasted_iota(jnp.int32, (1, 4), 1)
        yh_ref[...] = jnp.min(jnp.where(logits == mx, ids, 4), axis=1,
                              keepdims=True)


def _head(seg3, h, c1, c2, cp, p, clin):
    d4 = 4 * HID
    full_spec = lambda shape: pl.BlockSpec(shape, lambda i: tuple(0 for _ in shape))
    ins = [seg3, seg3, cp['W1'], cp['b1'].reshape(1, -1),
           cp['g1'].reshape(1, -1), cp['be1'].reshape(1, -1),
           cp['W2'], cp['b2'].reshape(1, -1),
           cp['lng'].reshape(1, -1), cp['lnb'].reshape(1, -1),
           h, c1, c2,
           p['Wphi'], p['bphi'].reshape(1, -1), p['Wa'], p['ba'].reshape(1, -1),
           p['Wb'], p['bb'].reshape(1, -1), p['Wc'].reshape(1, -1), p['bc'].reshape(1, 1),
           p['Wrho'], p['brho'].reshape(1, -1), p['Wwsi'], p['bwsi'].reshape(1, -1),
           p['Wo1'], p['bo1'].reshape(1, -1), p['Wo2'], p['bo2'].reshape(1, -1), clin]
    in_specs = [
        pl.BlockSpec((1, ROWS, HID), lambda i: (0, i, 0)),
        pl.BlockSpec((1, ROWS, HID), lambda i: (1, i, 0)),
        full_spec((HID, 2 * HID)), full_spec((1, 2 * HID)),
        full_spec((1, 2 * HID)), full_spec((1, 2 * HID)),
        full_spec((2 * HID, HID)), full_spec((1, HID)),
        full_spec((1, HID)), full_spec((1, HID)),
    ] + [pl.BlockSpec((ROWS, HID), lambda i: (i, 0))] * 3 + [
        full_spec((d4, d4)), full_spec((1, d4)),
        full_spec((d4, d4)), full_spec((1, d4)),
        full_spec((d4, d4)), full_spec((1, d4)),
        full_spec((1, d4)), full_spec((1, 1)),
        full_spec((d4, d4)), full_spec((1, d4)),
        full_spec((d4, 256)), full_spec((1, 256)),
        full_spec((288, HID)), full_spec((1, HID)),
        full_spec((HID, 4)), full_spec((1, 4)),
        full_spec((1, 32)),
    ]
    return pl.pallas_call(
        _head_body,
        grid=(GRID,),
        in_specs=in_specs,
        out_specs=[full_spec((1, 256)), full_spec((1, 4)), full_spec((1, 4)),
                   full_spec((1, 1))],
        out_shape=[jax.ShapeDtypeStruct((1, 256), jnp.float32),
                   jax.ShapeDtypeStruct((1, 4), jnp.float32),
                   jax.ShapeDtypeStruct((1, 4), jnp.float32),
                   jax.ShapeDtypeStruct((1, 1), jnp.int32)],
        scratch_shapes=[pltpu.SMEM((1, 1), jnp.float32),
                        pltpu.SMEM((1, 1), jnp.float32),
                        pltpu.VMEM((1, d4), jnp.float32)],
    )(*ins)


def kernel(x, edge_index, clin, params):
    p = params
    npad = EPAD - N_EDGES
    if npad:
        src = jnp.concatenate([edge_index[0], jnp.zeros((npad,), jnp.int32)])
        dst = jnp.concatenate([edge_index[1], jnp.full((npad,), DUMP, jnp.int32)])
    else:
        src = edge_index[0]
        dst = edge_index[1]

    h, qw = _fc_tables(x, p['Wfc'], p['bfc'], p['conv0']['t'])

    seg0 = _seg_call(qw.reshape(2 * N_NODES, HID), src, dst).reshape(2, NPAD, HID)
    cur1, qw1 = _mlp_layer(seg0, h, p['conv0'], has_norm=False, t_next=p['conv1']['t'])

    seg1 = _seg_call(qw1.reshape(2 * N_NODES, HID), src, dst).reshape(2, NPAD, HID)
    cur2, qw2 = _mlp_layer(seg1, cur1, p['conv1'], has_norm=True, t_next=p['conv2']['t'])

    seg2 = _seg_call(qw2.reshape(2 * N_NODES, HID), src, dst).reshape(2, NPAD, HID)
    wsi, hazards, S, yh = _head(seg2, h, cur1, cur2, p['conv2'], p, clin)
    return hazards, S, yh.reshape(1), wsi


# speedup vs baseline: 2.4928x; 1.0066x over previous
import functools

import jax
import jax.numpy as jnp
from jax import lax
from jax.experimental import pallas as pl
from jax.experimental.pallas import tpu as pltpu
from jax.experimental.pallas import tpu_sc as plsc

N_NODES = 10000
N_EDGES = 320000
HID = 128
EPS = 1e-7
DEN_EPS = 1e-16
LN_EPS = 1e-5

ROWS = 2000
GRID = N_NODES // ROWS


def _ln(h, g, b):
    mu = jnp.mean(h, axis=-1, keepdims=True)
    var = jnp.mean((h - mu) ** 2, axis=-1, keepdims=True)
    return (h - mu) / jnp.sqrt(var + LN_EPS) * g + b


def _tables(cur, t):
    y = jnp.maximum(cur, 0.0) + EPS
    w = jnp.exp(y * t)
    return y * w, w


def _fc_body(x_ref, w_ref, b_ref, t_ref, h_ref, qw_ref):
    h = jnp.dot(x_ref[...], w_ref[...], preferred_element_type=jnp.float32)
    h = jnp.maximum(h + b_ref[...], 0.0)
    h_ref[...] = h
    q, w = _tables(h, t_ref[...])
    qw_ref[0] = q
    qw_ref[1] = w


def _fc_tables(x, wfc, bfc, t):
    return pl.pallas_call(
        _fc_body,
        grid=(GRID,),
        in_specs=[
            pl.BlockSpec((ROWS, 1024), lambda i: (i, 0)),
            pl.BlockSpec((1024, HID), lambda i: (0, 0)),
            pl.BlockSpec((1, HID), lambda i: (0, 0)),
            pl.BlockSpec((1, 1), lambda i: (0, 0)),
        ],
        out_specs=[
            pl.BlockSpec((ROWS, HID), lambda i: (i, 0)),
            pl.BlockSpec((2, ROWS, HID), lambda i: (0, i, 0)),
        ],
        out_shape=[
            jax.ShapeDtypeStruct((N_NODES, HID), jnp.float32),
            jax.ShapeDtypeStruct((2, N_NODES, HID), jnp.float32),
        ],
    )(x, wfc, bfc.reshape(1, HID), t.reshape(1, 1))


NPAD = 10240


CH = 80
KB = 2
NS = 16
EPAD = N_EDGES
DUMP = 10200


def _build_seg():
    mesh = plsc.VectorSubcoreMesh(core_axis_name="c", subcore_axis_name="s")
    EPW = EPAD // NS
    CPT = EPW // CH
    NSUPER = CPT // KB
    RPS = NPAD // NS
    ZR = 32

    @functools.partial(
        pl.kernel,
        mesh=mesh,
        out_type=jax.ShapeDtypeStruct((2 * NPAD, HID), jnp.float32),
        scratch_types=[
            pltpu.VMEM((KB * CH,), jnp.int32),
            pltpu.VMEM((KB * CH,), jnp.int32),
            pltpu.VMEM((KB * CH,), jnp.int32),
            pltpu.VMEM((KB * CH,), jnp.int32),
            pltpu.VMEM((2 * KB, CH), jnp.int32),
            pltpu.VMEM((2 * KB, CH), jnp.int32),
            pltpu.VMEM((2, KB, CH, HID), jnp.float32),
            pltpu.VMEM((ZR, HID), jnp.float32),
            pltpu.VMEM_SHARED((NPAD, HID), jnp.float32),
            pltpu.SemaphoreType.DMA,
            pltpu.SemaphoreType.DMA,
            pltpu.SemaphoreType.DMA,
        ],
    )
    def seg(qw_hbm, src_hbm, dst_hbm, out_hbm, slin0, slin1, dlin0, dlin1,
            sblk, dblk, rows, buf, acc, isem, gsem, ssem):
        slin = (slin0, slin1)
        dlin = (dlin0, dlin1)
        c = lax.axis_index("c")
        s = lax.axis_index("s")
        coff = c * N_NODES
        ooff = c * NPAD

        def zrow(r, _):
            for j in range(HID // 16):
                buf[r, pl.ds(j * 16, 16)] = jnp.zeros((16,), jnp.float32)
            return _
        lax.fori_loop(0, ZR, zrow, None)

        def zacc(k, _):
            pltpu.sync_copy(buf, acc.at[pl.ds(s * RPS + k * ZR, ZR)])
            return _
        lax.fori_loop(0, RPS // ZR, zacc, None)
        plsc.subcore_barrier()

        e0 = s * EPW
        EB = KB * CH

        def load_idx(g, p):
            sl = pl.ds(e0 + g * EB, EB)
            pltpu.async_copy(src_hbm.at[sl], slin[p], isem)
            pltpu.async_copy(dst_hbm.at[sl], dlin[p], isem)

        def wait_idx(g, p):
            sl = pl.ds(e0 + g * EB, EB)
            pltpu.make_async_copy(src_hbm.at[sl], slin[p], isem).wait()
            pltpu.make_async_copy(dst_hbm.at[sl], dlin[p], isem).wait()

        def wait_scatters(p):
            for b in range(KB):
                r = p * KB + b
                pltpu.make_async_copy(rows.at[p, b], acc.at[dblk.at[r]],
                                      ssem).wait()

        def wait_gathers(p):
            for b in range(KB):
                pltpu.make_async_copy(qw_hbm.at[sblk.at[p * KB + b]],
                                      rows.at[p, b], gsem).wait()

        def fire_scatters(p):
            for b in range(KB):
                pltpu.async_copy(rows.at[p, b], acc.at[dblk.at[p * KB + b]],
                                 ssem, add=True)

        load_idx(0, 0)

        def block(i, g, p):
            q = 1 - p
            @pl.when(g >= 2)
            def _():
                wait_scatters(p)
            wait_idx(g, p)
            for b in range(KB):
                r = p * KB + b
                for k in range(CH // 16):
                    f = pl.ds(b * CH + k * 16, 16)
                    sl = pl.ds(k * 16, 16)
                    sblk[r, sl] = slin[p][f] + coff
                    dblk[r, sl] = dlin[p][f]
            @pl.when(g + 1 < NSUPER)
            def _():
                load_idx(g + 1, q)
            for b in range(KB):
                pltpu.async_copy(qw_hbm.at[sblk.at[p * KB + b]],
                                 rows.at[p, b], gsem)
            @pl.when(g >= 1)
            def _():
                wait_gathers(q)
                fire_scatters(q)

        def two_blocks(i, _):
            block(i, 2 * i, 0)
            block(i, 2 * i + 1, 1)
            return _
        lax.fori_loop(0, NSUPER // 2, two_blocks, None)
        if NSUPER % 2:
            block(0, jnp.int32(NSUPER - 1), 0)
        last = (NSUPER - 1) % 2
        wait_gathers(last)
        fire_scatters(last)
        wait_scatters(1 - last)
        wait_scatters(last)

        plsc.subcore_barrier()

        r0 = s * RPS
        pltpu.sync_copy(acc.at[pl.ds(r0, RPS)], out_hbm.at[pl.ds(ooff + r0, RPS)])

    return seg


_seg_cache = []


def _seg_call(qw2, src, dst):
    if not _seg_cache:
        _seg_cache.append(_build_seg())
    return _seg_cache[0](qw2, src, dst)


def _mlp_body(has_norm, make_tables,
              num_ref, den_ref, cur_ref, w1_ref, b1_ref, g1_ref, be1_ref,
              w2_ref, b2_ref, *rest):
    if has_norm:
        lng_ref, lnb_ref = rest[0], rest[1]
        rest = rest[2:]
    if make_tables:
        t_ref = rest[0]
        rest = rest[1:]
    cur_out_ref = rest[0]

    cur = cur_ref[...]
    agg = num_ref[0] / (den_ref[0] + DEN_EPS) + cur
    h1 = jnp.dot(agg, w1_ref[...], preferred_element_type=jnp.float32) + b1_ref[...]
    h1 = jnp.maximum(_ln(h1, g1_ref[...], be1_ref[...]), 0.0)
    co = jnp.dot(h1, w2_ref[...], preferred_element_type=jnp.float32) + b2_ref[...]
    if has_norm:
        co = cur + jnp.maximum(_ln(co, lng_ref[...], lnb_ref[...]), 0.0)
    cur_out_ref[...] = co
    if make_tables:
        qw_ref = rest[1]
        q, w = _tables(co, t_ref[...])
        qw_ref[0] = q
        qw_ref[1] = w


def _mlp_layer(seg3, cur, c, has_norm, t_next):
    make_tables = t_next is not None
    ins = [seg3, seg3, cur, c['W1'], c['b1'].reshape(1, -1),
           c['g1'].reshape(1, -1), c['be1'].reshape(1, -1),
           c['W2'], c['b2'].reshape(1, -1)]
    in_specs = [
        pl.BlockSpec((1, ROWS, HID), lambda i: (0, i, 0)),
        pl.BlockSpec((1, ROWS, HID), lambda i: (1, i, 0)),
        pl.BlockSpec((ROWS, HID), lambda i: (i, 0)),
        pl.BlockSpec((HID, 2 * HID), lambda i: (0, 0)),
        pl.BlockSpec((1, 2 * HID), lambda i: (0, 0)),
        pl.BlockSpec((1, 2 * HID), lambda i: (0, 0)),
        pl.BlockSpec((1, 2 * HID), lambda i: (0, 0)),
        pl.BlockSpec((2 * HID, HID), lambda i: (0, 0)),
        pl.BlockSpec((1, HID), lambda i: (0, 0)),
    ]
    if has_norm:
        ins += [c['lng'].reshape(1, -1), c['lnb'].reshape(1, -1)]
        in_specs += [pl.BlockSpec((1, HID), lambda i: (0, 0)),
                     pl.BlockSpec((1, HID), lambda i: (0, 0))]
    if make_tables:
        ins += [t_next.reshape(1, 1)]
        in_specs += [pl.BlockSpec((1, 1), lambda i: (0, 0))]

    out_specs = [pl.BlockSpec((ROWS, HID), lambda i: (i, 0))]
    out_shape = [jax.ShapeDtypeStruct((N_NODES, HID), jnp.float32)]
    if make_tables:
        out_specs.append(pl.BlockSpec((2, ROWS, HID), lambda i: (0, i, 0)))
        out_shape.append(jax.ShapeDtypeStruct((2, N_NODES, HID), jnp.float32))

    res = pl.pallas_call(
        functools.partial(_mlp_body, has_norm, make_tables),
        grid=(GRID,),
        in_specs=in_specs,
        out_specs=out_specs,
        out_shape=out_shape,
    )(*ins)
    return res if make_tables else (res[0], None)


def _head_body(num_ref, den_ref, w1_ref, b1_ref, g1_ref, be1_ref,
               w2_ref, b2_ref, lng_ref, lnb_ref,
               h_ref, c1_ref, c2_ref,
               wphi_ref, bphi_ref, wa_ref, ba_ref, wb_ref, bb_ref,
               wc_ref, bc_ref, wrho_ref, brho_ref, wwsi_ref, bwsi_ref,
               wo1_ref, bo1_ref, wo2_ref, bo2_ref, clin_ref,
               wsi_ref, haz_ref, ss_ref, yh_ref, m_ref, s_ref, acc_ref):
    i = pl.program_id(0)

    @pl.when(i == 0)
    def _init():
        m_ref[0, 0] = -1e30
        s_ref[0, 0] = 0.0
        acc_ref[...] = jnp.zeros_like(acc_ref)

    cur = c2_ref[...]
    agg = num_ref[0] / (den_ref[0] + DEN_EPS) + cur
    h1 = jnp.dot(agg, w1_ref[...], preferred_element_type=jnp.float32) + b1_ref[...]
    h1 = jnp.maximum(_ln(h1, g1_ref[...], be1_ref[...]), 0.0)
    co = jnp.dot(h1, w2_ref[...], preferred_element_type=jnp.float32) + b2_ref[...]
    c3 = cur + jnp.maximum(_ln(co, lng_ref[...], lnb_ref[...]), 0.0)

    xb = jnp.concatenate([h_ref[...], c1_ref[...], cur, c3], axis=1)
    hp = jnp.dot(xb, wphi_ref[...], preferred_element_type=jnp.float32)
    hp = jnp.maximum(hp + bphi_ref[...], 0.0)
    a = jnp.tanh(jnp.dot(hp, wa_ref[...], preferred_element_type=jnp.float32) + ba_ref[...])
    b = jax.nn.sigmoid(jnp.dot(hp, wb_ref[...], preferred_element_type=jnp.float32) + bb_ref[...])
    att = jnp.sum((a * b) * wc_ref[...], axis=1, keepdims=True) + bc_ref[...]

    m_old = m_ref[0, 0]
    m_new = jnp.maximum(m_old, jnp.max(att))
    corr = jnp.exp(m_old - m_new)
    p = jnp.exp(att - m_new)
    s_new = s_ref[0, 0] * corr + jnp.sum(p)
    acc_new = acc_ref[...] * corr + jnp.sum(p * hp, axis=0, keepdims=True)
    m_ref[0, 0] = m_new
    s_ref[0, 0] = s_new
    acc_ref[...] = acc_new

    @pl.when(i == GRID - 1)
    def _tail():
        hpool = acc_new / s_new
        hr = jnp.maximum(jnp.dot(hpool, wrho_ref[...], preferred_element_type=jnp.float32)
                         + brho_ref[...], 0.0)
        wsi = jnp.maximum(jnp.dot(hr, wwsi_ref[...], preferred_element_type=jnp.float32)
                          + bwsi_ref[...], 0.0)
        wsi_ref[...] = wsi
        full = jnp.concatenate([wsi, clin_ref[...]], axis=1)
        l1 = jnp.maximum(jnp.dot(full, wo1_ref[...], preferred_element_type=jnp.float32)
                         + bo1_ref[...], 0.0)
        logits = jnp.dot(l1, wo2_ref[...], preferred_element_type=jnp.float32) \
            + bo2_ref[...]
        hz = jax.nn.sigmoid(logits)
        haz_ref[...] = hz
        q = 1.0 - hz
        s0 = q[:, 0:1]
        s1 = s0 * q[:, 1:2]
        s2 = s1 * q[:, 2:3]
        s3 = s2 * q[:, 3:4]
        ss_ref[...] = jnp.concatenate([s0, s1, s2, s3], axis=1)
        mx = jnp.max(logits)
        ids = lax.broadcasted_iota(jnp.int32, (1, 4), 1)
        yh_ref[...] = jnp.min(jnp.where(logits == mx, ids, 4), axis=1,
                              keepdims=True)


def _head(seg3, h, c1, c2, cp, p, clin):
    d4 = 4 * HID
    full_spec = lambda shape: pl.BlockSpec(shape, lambda i: tuple(0 for _ in shape))
    ins = [seg3, seg3, cp['W1'], cp['b1'].reshape(1, -1),
           cp['g1'].reshape(1, -1), cp['be1'].reshape(1, -1),
           cp['W2'], cp['b2'].reshape(1, -1),
           cp['lng'].reshape(1, -1), cp['lnb'].reshape(1, -1),
           h, c1, c2,
           p['Wphi'], p['bphi'].reshape(1, -1), p['Wa'], p['ba'].reshape(1, -1),
           p['Wb'], p['bb'].reshape(1, -1), p['Wc'].reshape(1, -1), p['bc'].reshape(1, 1),
           p['Wrho'], p['brho'].reshape(1, -1), p['Wwsi'], p['bwsi'].reshape(1, -1),
           p['Wo1'], p['bo1'].reshape(1, -1), p['Wo2'], p['bo2'].reshape(1, -1), clin]
    in_specs = [
        pl.BlockSpec((1, ROWS, HID), lambda i: (0, i, 0)),
        pl.BlockSpec((1, ROWS, HID), lambda i: (1, i, 0)),
        full_spec((HID, 2 * HID)), full_spec((1, 2 * HID)),
        full_spec((1, 2 * HID)), full_spec((1, 2 * HID)),
        full_spec((2 * HID, HID)), full_spec((1, HID)),
        full_spec((1, HID)), full_spec((1, HID)),
    ] + [pl.BlockSpec((ROWS, HID), lambda i: (i, 0))] * 3 + [
        full_spec((d4, d4)), full_spec((1, d4)),
        full_spec((d4, d4)), full_spec((1, d4)),
        full_spec((d4, d4)), full_spec((1, d4)),
        full_spec((1, d4)), full_spec((1, 1)),
        full_spec((d4, d4)), full_spec((1, d4)),
        full_spec((d4, 256)), full_spec((1, 256)),
        full_spec((288, HID)), full_spec((1, HID)),
        full_spec((HID, 4)), full_spec((1, 4)),
        full_spec((1, 32)),
    ]
    return pl.pallas_call(
        _head_body,
        grid=(GRID,),
        in_specs=in_specs,
        out_specs=[full_spec((1, 256)), full_spec((1, 4)), full_spec((1, 4)),
                   full_spec((1, 1))],
        out_shape=[jax.ShapeDtypeStruct((1, 256), jnp.float32),
                   jax.ShapeDtypeStruct((1, 4), jnp.float32),
                   jax.ShapeDtypeStruct((1, 4), jnp.float32),
                   jax.ShapeDtypeStruct((1, 1), jnp.int32)],
        scratch_shapes=[pltpu.SMEM((1, 1), jnp.float32),
                        pltpu.SMEM((1, 1), jnp.float32),
                        pltpu.VMEM((1, d4), jnp.float32)],
    )(*ins)


def kernel(x, edge_index, clin, params):
    p = params
    npad = EPAD - N_EDGES
    if npad:
        src = jnp.concatenate([edge_index[0], jnp.zeros((npad,), jnp.int32)])
        dst = jnp.concatenate([edge_index[1], jnp.full((npad,), DUMP, jnp.int32)])
    else:
        src = edge_index[0]
        dst = edge_index[1]

    h, qw = _fc_tables(x, p['Wfc'], p['bfc'], p['conv0']['t'])

    seg0 = _seg_call(qw.reshape(2 * N_NODES, HID), src, dst).reshape(2, NPAD, HID)
    cur1, qw1 = _mlp_layer(seg0, h, p['conv0'], has_norm=False, t_next=p['conv1']['t'])

    seg1 = _seg_call(qw1.reshape(2 * N_NODES, HID), src, dst).reshape(2, NPAD, HID)
    cur2, qw2 = _mlp_layer(seg1, cur1, p['conv1'], has_norm=True, t_next=p['conv2']['t'])

    seg2 = _seg_call(qw2.reshape(2 * N_NODES, HID), src, dst).reshape(2, NPAD, HID)
    wsi, hazards, S, yh = _head(seg2, h, cur1, cur2, p['conv2'], p, clin)
    return hazards, S, yh.reshape(1), wsi
